# Initial kernel scaffold; baseline (speedup 1.0000x reference)
import jax
import jax.numpy as jnp
from jax.experimental import pallas as pl


def kernel(pt_coordinates, centroids):
    raise NotImplementedError("write your pallas kernel here")


# speedup vs baseline: 16.7930x; 16.7930x over previous
import jax
import jax.numpy as jnp
import numpy as np
from jax import lax
from jax.experimental import pallas as pl
from jax.experimental.pallas import tpu as pltpu
from jax.experimental.pallas import tpu_sc as plsc

_B = 8
_N = 8192
_M = 1024
_K = 64
_RSQ = np.float32(0.2 * 0.2)

_L = 16
_U = 8
_BLK = _U * _L
_NBLK = _N // _BLK
_ROW = _K + _BLK
_WPB = 4
_CPW = _M // _WPB


def _sc_body(pt_hbm, cen_hbm, out_hbm, px, py, pz, sqp, cx, cy, cz, row, res):
    wid = lax.axis_index("s") * 2 + lax.axis_index("c")
    b = wid // _WPB
    q = wid % _WPB

    pltpu.sync_copy(pt_hbm.at[pl.ds((b * 3 + 0) * _N, _N)], px)
    pltpu.sync_copy(pt_hbm.at[pl.ds((b * 3 + 1) * _N, _N)], py)
    pltpu.sync_copy(pt_hbm.at[pl.ds((b * 3 + 2) * _N, _N)], pz)
    pltpu.sync_copy(cen_hbm.at[pl.ds((b * 3 + 0) * _M + q * _CPW, _CPW)], cx)
    pltpu.sync_copy(cen_hbm.at[pl.ds((b * 3 + 1) * _M + q * _CPW, _CPW)], cy)
    pltpu.sync_copy(cen_hbm.at[pl.ds((b * 3 + 2) * _M + q * _CPW, _CPW)], cz)

    iota = lax.iota(jnp.int32, _L)
    zeros16 = jnp.zeros((_L,), jnp.int32)
    sentinel = jnp.full((_L,), _N, jnp.int32)

    def _rne_bf16(v):
        u = plsc.bitcast(v, jnp.uint32)
        lsb = (u >> 16) & jnp.uint32(1)
        r = (u + jnp.uint32(0x7FFF) + lsb) & jnp.uint32(0xFFFF0000)
        return plsc.bitcast(r, jnp.float32)

    def precomp(j, carry):
        s = pl.ds(j * _L, _L)
        vx = px[s]
        vy = py[s]
        vz = pz[s]
        sqp[s] = (vx * vx + vy * vy) + vz * vz
        px[s] = _rne_bf16(vx)
        py[s] = _rne_bf16(vy)
        pz[s] = _rne_bf16(vz)
        return carry

    lax.fori_loop(0, _N // _L, precomp, 0)

    def per_centroid(c, carry):
        cidx = jnp.full((_L,), c, jnp.int32)
        vcx = plsc.load_gather(cx, [cidx])
        vcy = plsc.load_gather(cy, [cidx])
        vcz = plsc.load_gather(cz, [cidx])
        sqc = (vcx * vcx + vcy * vcy) + vcz * vcz
        vcx = _rne_bf16(vcx)
        vcy = _rne_bf16(vcy)
        vcz = _rne_bf16(vcz)

        def cond(st):
            i, _, ps = st
            return (i < _NBLK) & (ps < _K)

        def body(st):
            i, ptrv, _ = st
            base0 = i * _BLK
            for u in range(_U):
                base = base0 + u * _L
                s = pl.ds(base, _L)
                vx = px[s]
                vy = py[s]
                vz = pz[s]
                vsqp = sqp[s]
                cross = (vcx * vx + vcy * vy) + vcz * vz
                d2 = (sqc + vsqp) - 2.0 * cross
                mask = d2 <= _RSQ
                prefix = plsc.cumsum(mask.astype(jnp.int32))
                pos = (ptrv + prefix) - 1
                plsc.store_scatter(row, [pos], iota + base, mask=mask)
                ptrv = ptrv + plsc.all_reduce_population_count(mask)
            return i + 1, ptrv, jnp.max(ptrv)

        _, ptrv, _ = lax.while_loop(
            cond, body, (jnp.int32(0), zeros16, jnp.int32(0))
        )

        row0 = row[pl.ds(0, _L)]
        first_s = jnp.min(jnp.where(iota == 0, row0, jnp.int32(2 ** 30)))
        padv = jnp.where(ptrv > 0, jnp.full((_L,), first_s, jnp.int32), sentinel)
        for qq in range(_K // _L):
            lanes = iota + qq * _L
            vals = row[pl.ds(qq * _L, _L)]
            res[pl.ds(c * _K + qq * _L, _L)] = jnp.where(lanes < ptrv, vals, padv)
        return carry

    lax.fori_loop(0, _CPW, per_centroid, 0)

    pltpu.sync_copy(res, out_hbm.at[pl.ds((b * _M + q * _CPW) * _K, _CPW * _K)])


@jax.jit
def kernel(pt_coordinates, centroids):
    f = pl.kernel(
        _sc_body,
        out_type=jax.ShapeDtypeStruct((_B * _M * _K,), jnp.int32),
        mesh=plsc.VectorSubcoreMesh(core_axis_name="c", subcore_axis_name="s"),
        compiler_params=pltpu.CompilerParams(needs_layout_passes=False),
        scratch_types=[
            pltpu.VMEM((_N,), jnp.float32),
            pltpu.VMEM((_N,), jnp.float32),
            pltpu.VMEM((_N,), jnp.float32),
            pltpu.VMEM((_N,), jnp.float32),
            pltpu.VMEM((_CPW,), jnp.float32),
            pltpu.VMEM((_CPW,), jnp.float32),
            pltpu.VMEM((_CPW,), jnp.float32),
            pltpu.VMEM((_ROW,), jnp.int32),
            pltpu.VMEM((_CPW * _K,), jnp.int32),
        ],
    )
    out = f(pt_coordinates.reshape(-1), centroids.reshape(-1))
    return out.reshape(_B, _M, _K)


# speedup vs baseline: 51.6807x; 3.0775x over previous
import jax
import jax.numpy as jnp
import numpy as np
from jax import lax
from jax.experimental import pallas as pl
from jax.experimental.pallas import tpu as pltpu
from jax.experimental.pallas import tpu_sc as plsc

_B = 8
_N = 8192
_M = 1024
_K = 64
_RSQ = np.float32(0.2 * 0.2)

_L = 16
_U = 8
_BLK = _U * _L
_NBLK = _N // _BLK
_ROW = _K + _BLK
_WPB = 4
_CPW = _M // _WPB


def _sc_body(pt_hbm, cen_hbm, out_hbm, px, py, pz, sqp, cx, cy, cz, row, res):
    wid = lax.axis_index("s") * 2 + lax.axis_index("c")
    b = wid // _WPB
    q = wid % _WPB

    pltpu.sync_copy(pt_hbm.at[pl.ds((b * 3 + 0) * _N, _N)], px)
    pltpu.sync_copy(pt_hbm.at[pl.ds((b * 3 + 1) * _N, _N)], py)
    pltpu.sync_copy(pt_hbm.at[pl.ds((b * 3 + 2) * _N, _N)], pz)
    pltpu.sync_copy(cen_hbm.at[pl.ds((b * 3 + 0) * _M + q * _CPW, _CPW)], cx)
    pltpu.sync_copy(cen_hbm.at[pl.ds((b * 3 + 1) * _M + q * _CPW, _CPW)], cy)
    pltpu.sync_copy(cen_hbm.at[pl.ds((b * 3 + 2) * _M + q * _CPW, _CPW)], cz)

    iota = lax.iota(jnp.int32, _L)
    zeros16 = jnp.zeros((_L,), jnp.int32)
    sentinel = jnp.full((_L,), _N, jnp.int32)

    def _rne_bf16(v):
        u = plsc.bitcast(v, jnp.uint32)
        lsb = (u >> 16) & jnp.uint32(1)
        r = (u + jnp.uint32(0x7FFF) + lsb) & jnp.uint32(0xFFFF0000)
        return plsc.bitcast(r, jnp.float32)

    def precomp(j, carry):
        s = pl.ds(j * _L, _L)
        vx = px[s]
        vy = py[s]
        vz = pz[s]
        sqp[s] = (vx * vx + vy * vy) + vz * vz
        px[s] = _rne_bf16(vx)
        py[s] = _rne_bf16(vy)
        pz[s] = _rne_bf16(vz)
        return carry

    lax.fori_loop(0, _N // _L, precomp, 0)

    def per_centroid(c, carry):
        cidx = jnp.full((_L,), c, jnp.int32)
        vcx = plsc.load_gather(cx, [cidx])
        vcy = plsc.load_gather(cy, [cidx])
        vcz = plsc.load_gather(cz, [cidx])
        sqc = (vcx * vcx + vcy * vcy) + vcz * vcz
        vcx = _rne_bf16(vcx)
        vcy = _rne_bf16(vcy)
        vcz = _rne_bf16(vcz)

        def cond(st):
            i, _, ps = st
            return (i < _NBLK) & (ps < _K)

        def body(st):
            i, ptrv, _ = st
            base0 = i * _BLK
            masks = []
            prefixes = []
            counts = []
            for u in range(_U):
                s = pl.ds(base0 + u * _L, _L)
                vx = px[s]
                vy = py[s]
                vz = pz[s]
                vsqp = sqp[s]
                cross = (vcx * vx + vcy * vy) + vcz * vz
                d2 = (sqc + vsqp) - 2.0 * cross
                mask = d2 <= _RSQ
                masks.append(mask)
                prefixes.append(plsc.cumsum(mask.astype(jnp.int32)))
                counts.append(plsc.all_reduce_population_count(mask))
            for u in range(_U):
                pos = (ptrv + prefixes[u]) - 1
                plsc.store_scatter(row, [pos], iota + (base0 + u * _L),
                                   mask=masks[u])
                ptrv = ptrv + counts[u]
            return i + 1, ptrv, jnp.max(ptrv)

        _, ptrv, _ = lax.while_loop(
            cond, body, (jnp.int32(0), zeros16, jnp.int32(0))
        )

        row0 = row[pl.ds(0, _L)]
        first_s = jnp.min(jnp.where(iota == 0, row0, jnp.int32(2 ** 30)))
        padv = jnp.where(ptrv > 0, jnp.full((_L,), first_s, jnp.int32), sentinel)
        for qq in range(_K // _L):
            lanes = iota + qq * _L
            vals = row[pl.ds(qq * _L, _L)]
            res[pl.ds(c * _K + qq * _L, _L)] = jnp.where(lanes < ptrv, vals, padv)
        return carry

    lax.fori_loop(0, _CPW, per_centroid, 0)

    pltpu.sync_copy(res, out_hbm.at[pl.ds((b * _M + q * _CPW) * _K, _CPW * _K)])


@jax.jit
def kernel(pt_coordinates, centroids):
    f = pl.kernel(
        _sc_body,
        out_type=jax.ShapeDtypeStruct((_B * _M * _K,), jnp.int32),
        mesh=plsc.VectorSubcoreMesh(core_axis_name="c", subcore_axis_name="s"),
        compiler_params=pltpu.CompilerParams(needs_layout_passes=False),
        scratch_types=[
            pltpu.VMEM((_N,), jnp.float32),
            pltpu.VMEM((_N,), jnp.float32),
            pltpu.VMEM((_N,), jnp.float32),
            pltpu.VMEM((_N,), jnp.float32),
            pltpu.VMEM((_CPW,), jnp.float32),
            pltpu.VMEM((_CPW,), jnp.float32),
            pltpu.VMEM((_CPW,), jnp.float32),
            pltpu.VMEM((_ROW,), jnp.int32),
            pltpu.VMEM((_CPW * _K,), jnp.int32),
        ],
    )
    out = f(pt_coordinates.reshape(-1), centroids.reshape(-1))
    return out.reshape(_B, _M, _K)


# speedup vs baseline: 62.9843x; 1.2187x over previous
import jax
import jax.numpy as jnp
import numpy as np
from jax import lax
from jax.experimental import pallas as pl
from jax.experimental.pallas import tpu as pltpu
from jax.experimental.pallas import tpu_sc as plsc

_B = 8
_N = 8192
_M = 1024
_K = 64
_RSQ = np.float32(0.2 * 0.2)

_L = 16
_U = 16
_BLK = _U * _L
_NBLK = _N // _BLK
_ROW = _K + _BLK
_WPB = 4
_CPW = _M // _WPB


def _sc_body(pt_hbm, cen_hbm, out_hbm, px, py, pz, sqp, cx, cy, cz, row, res):
    wid = lax.axis_index("s") * 2 + lax.axis_index("c")
    b = wid // _WPB
    q = wid % _WPB

    pltpu.sync_copy(pt_hbm.at[pl.ds((b * 3 + 0) * _N, _N)], px)
    pltpu.sync_copy(pt_hbm.at[pl.ds((b * 3 + 1) * _N, _N)], py)
    pltpu.sync_copy(pt_hbm.at[pl.ds((b * 3 + 2) * _N, _N)], pz)
    pltpu.sync_copy(cen_hbm.at[pl.ds((b * 3 + 0) * _M + q * _CPW, _CPW)], cx)
    pltpu.sync_copy(cen_hbm.at[pl.ds((b * 3 + 1) * _M + q * _CPW, _CPW)], cy)
    pltpu.sync_copy(cen_hbm.at[pl.ds((b * 3 + 2) * _M + q * _CPW, _CPW)], cz)

    iota = lax.iota(jnp.int32, _L)
    zeros16 = jnp.zeros((_L,), jnp.int32)
    sentinel = jnp.full((_L,), _N, jnp.int32)

    def _rne_bf16(v):
        u = plsc.bitcast(v, jnp.uint32)
        lsb = (u >> 16) & jnp.uint32(1)
        r = (u + jnp.uint32(0x7FFF) + lsb) & jnp.uint32(0xFFFF0000)
        return plsc.bitcast(r, jnp.float32)

    def precomp(j, carry):
        s = pl.ds(j * _L, _L)
        vx = px[s]
        vy = py[s]
        vz = pz[s]
        sqp[s] = (vx * vx + vy * vy) + vz * vz
        px[s] = _rne_bf16(vx)
        py[s] = _rne_bf16(vy)
        pz[s] = _rne_bf16(vz)
        return carry

    lax.fori_loop(0, _N // _L, precomp, 0)

    def per_centroid(c, carry):
        cidx = jnp.full((_L,), c, jnp.int32)
        vcx = plsc.load_gather(cx, [cidx])
        vcy = plsc.load_gather(cy, [cidx])
        vcz = plsc.load_gather(cz, [cidx])
        sqc = (vcx * vcx + vcy * vcy) + vcz * vcz
        vcx = _rne_bf16(vcx)
        vcy = _rne_bf16(vcy)
        vcz = _rne_bf16(vcz)

        def cond(st):
            i, _, ps = st
            return (i < _NBLK) & (ps < _K)

        def body(st):
            i, ptrv, _ = st
            base0 = i * _BLK
            masks = []
            prefixes = []
            counts = []
            for u in range(_U):
                s = pl.ds(base0 + u * _L, _L)
                vx = px[s]
                vy = py[s]
                vz = pz[s]
                vsqp = sqp[s]
                cross = (vcx * vx + vcy * vy) + vcz * vz
                d2 = (sqc + vsqp) - 2.0 * cross
                mask = d2 <= _RSQ
                masks.append(mask)
                prefixes.append(plsc.cumsum(mask.astype(jnp.int32)))
                counts.append(plsc.all_reduce_population_count(mask))
            for u in range(_U):
                pos = (ptrv + prefixes[u]) - 1
                plsc.store_scatter(row, [pos], iota + (base0 + u * _L),
                                   mask=masks[u])
                ptrv = ptrv + counts[u]
            return i + 1, ptrv, jnp.max(ptrv)

        _, ptrv, _ = lax.while_loop(
            cond, body, (jnp.int32(0), zeros16, jnp.int32(0))
        )

        row0 = row[pl.ds(0, _L)]
        first_s = jnp.min(jnp.where(iota == 0, row0, jnp.int32(2 ** 30)))
        padv = jnp.where(ptrv > 0, jnp.full((_L,), first_s, jnp.int32), sentinel)
        for qq in range(_K // _L):
            lanes = iota + qq * _L
            vals = row[pl.ds(qq * _L, _L)]
            res[pl.ds(c * _K + qq * _L, _L)] = jnp.where(lanes < ptrv, vals, padv)
        return carry

    lax.fori_loop(0, _CPW, per_centroid, 0)

    pltpu.sync_copy(res, out_hbm.at[pl.ds((b * _M + q * _CPW) * _K, _CPW * _K)])


@jax.jit
def kernel(pt_coordinates, centroids):
    f = pl.kernel(
        _sc_body,
        out_type=jax.ShapeDtypeStruct((_B * _M * _K,), jnp.int32),
        mesh=plsc.VectorSubcoreMesh(core_axis_name="c", subcore_axis_name="s"),
        compiler_params=pltpu.CompilerParams(needs_layout_passes=False),
        scratch_types=[
            pltpu.VMEM((_N,), jnp.float32),
            pltpu.VMEM((_N,), jnp.float32),
            pltpu.VMEM((_N,), jnp.float32),
            pltpu.VMEM((_N,), jnp.float32),
            pltpu.VMEM((_CPW,), jnp.float32),
            pltpu.VMEM((_CPW,), jnp.float32),
            pltpu.VMEM((_CPW,), jnp.float32),
            pltpu.VMEM((_ROW,), jnp.int32),
            pltpu.VMEM((_CPW * _K,), jnp.int32),
        ],
    )
    out = f(pt_coordinates.reshape(-1), centroids.reshape(-1))
    return out.reshape(_B, _M, _K)


# speedup vs baseline: 67.0380x; 1.0644x over previous
import jax
import jax.numpy as jnp
import numpy as np
from jax import lax
from jax.experimental import pallas as pl
from jax.experimental.pallas import tpu as pltpu
from jax.experimental.pallas import tpu_sc as plsc

_B = 8
_N = 8192
_M = 1024
_K = 64
_RSQ = np.float32(0.2 * 0.2)

_L = 16
_U = 16
_BLK = _U * _L
_NBLK = _N // _BLK
_ROW = _K + _BLK
_WPB = 4
_CPW = _M // _WPB


def _sc_body(pt_hbm, cen_hbm, out_hbm, px, py, pz, cx, cy, cz, row, res):
    wid = lax.axis_index("s") * 2 + lax.axis_index("c")
    b = wid // _WPB
    q = wid % _WPB

    pltpu.sync_copy(pt_hbm.at[pl.ds((b * 3 + 0) * _N, _N)], px)
    pltpu.sync_copy(pt_hbm.at[pl.ds((b * 3 + 1) * _N, _N)], py)
    pltpu.sync_copy(pt_hbm.at[pl.ds((b * 3 + 2) * _N, _N)], pz)
    pltpu.sync_copy(cen_hbm.at[pl.ds((b * 3 + 0) * _M + q * _CPW, _CPW)], cx)
    pltpu.sync_copy(cen_hbm.at[pl.ds((b * 3 + 1) * _M + q * _CPW, _CPW)], cy)
    pltpu.sync_copy(cen_hbm.at[pl.ds((b * 3 + 2) * _M + q * _CPW, _CPW)], cz)

    iota = lax.iota(jnp.int32, _L)
    sentinel = jnp.full((_L,), _N, jnp.int32)

    def _rne_bf16(v):
        u = plsc.bitcast(v, jnp.uint32)
        lsb = (u >> 16) & jnp.uint32(1)
        r = (u + jnp.uint32(0x7FFF) + lsb) & jnp.uint32(0xFFFF0000)
        return plsc.bitcast(r, jnp.float32)

    def precomp(j, carry):
        s = pl.ds(j * _L, _L)
        vx = px[s]
        vy = py[s]
        vz = pz[s]
        vsqp = (vx * vx + vy * vy) + vz * vz
        xb = plsc.bitcast(_rne_bf16(vx) * 2.0, jnp.uint32)
        yb = plsc.bitcast(_rne_bf16(vy) * 2.0, jnp.uint32)
        px[s] = plsc.bitcast(xb | (yb >> 16), jnp.float32)
        py[s] = _rne_bf16(vz) * 2.0
        pz[s] = vsqp
        return carry

    lax.fori_loop(0, _N // _L, precomp, 0)
    ones16 = jnp.ones((_L,), jnp.int32)

    def per_centroid(c, carry):
        cidx = jnp.full((_L,), c, jnp.int32)
        vcx = plsc.load_gather(cx, [cidx])
        vcy = plsc.load_gather(cy, [cidx])
        vcz = plsc.load_gather(cz, [cidx])
        sqc = (vcx * vcx + vcy * vcy) + vcz * vcz
        vcx = _rne_bf16(vcx)
        vcy = _rne_bf16(vcy)
        vcz = _rne_bf16(vcz)

        def cond(st):
            i, _, ps = st
            return (i < _NBLK) & (ps < _K)

        def body(st):
            i, ptrv, _ = st
            base0 = i * _BLK
            masks = []
            prefixes = []
            counts = []
            for u in range(_U):
                s = pl.ds(base0 + u * _L, _L)
                w = plsc.bitcast(px[s], jnp.uint32)
                vx2 = plsc.bitcast(w & jnp.uint32(0xFFFF0000), jnp.float32)
                vy2 = plsc.bitcast(w << 16, jnp.float32)
                vz2 = py[s]
                vsqp = pz[s]
                cross2 = (vcx * vx2 + vcy * vy2) + vcz * vz2
                d2 = (sqc + vsqp) - cross2
                mask = d2 <= _RSQ
                masks.append(mask)
                prefixes.append(plsc.cumsum(ones16, mask=mask))
                counts.append(plsc.all_reduce_population_count(mask))
            for u in range(_U):
                pos = (ptrv + prefixes[u]) - 1
                plsc.store_scatter(row, [pos], iota + (base0 + u * _L),
                                   mask=masks[u])
                ptrv = ptrv + counts[u]
            return i + 1, ptrv, jnp.max(ptrv)

        _, ptrv, _ = lax.while_loop(
            cond, body, (jnp.int32(0), jnp.zeros((_L,), jnp.int32), jnp.int32(0))
        )

        row0 = row[pl.ds(0, _L)]
        first_s = jnp.min(jnp.where(iota == 0, row0, jnp.int32(2 ** 30)))
        padv = jnp.where(ptrv > 0, jnp.full((_L,), first_s, jnp.int32), sentinel)
        for qq in range(_K // _L):
            lanes = iota + qq * _L
            vals = row[pl.ds(qq * _L, _L)]
            res[pl.ds(c * _K + qq * _L, _L)] = jnp.where(lanes < ptrv, vals, padv)
        return carry

    lax.fori_loop(0, _CPW, per_centroid, 0)

    pltpu.sync_copy(res, out_hbm.at[pl.ds((b * _M + q * _CPW) * _K, _CPW * _K)])


@jax.jit
def kernel(pt_coordinates, centroids):
    f = pl.kernel(
        _sc_body,
        out_type=jax.ShapeDtypeStruct((_B * _M * _K,), jnp.int32),
        mesh=plsc.VectorSubcoreMesh(core_axis_name="c", subcore_axis_name="s"),
        compiler_params=pltpu.CompilerParams(needs_layout_passes=False),
        scratch_types=[
            pltpu.VMEM((_N,), jnp.float32),
            pltpu.VMEM((_N,), jnp.float32),
            pltpu.VMEM((_N,), jnp.float32),
            pltpu.VMEM((_CPW,), jnp.float32),
            pltpu.VMEM((_CPW,), jnp.float32),
            pltpu.VMEM((_CPW,), jnp.float32),
            pltpu.VMEM((_ROW,), jnp.int32),
            pltpu.VMEM((_CPW * _K,), jnp.int32),
        ],
    )
    out = f(pt_coordinates.reshape(-1), centroids.reshape(-1))
    return out.reshape(_B, _M, _K)


# speedup vs baseline: 67.0497x; 1.0002x over previous
import jax
import jax.numpy as jnp
import numpy as np
from jax import lax
from jax.experimental import pallas as pl
from jax.experimental.pallas import tpu as pltpu
from jax.experimental.pallas import tpu_sc as plsc

_B = 8
_N = 8192
_M = 1024
_K = 64
_RSQ = np.float32(0.2 * 0.2)

_L = 16
_U = 16
_BLK = _U * _L
_NBLK = _N // _BLK
_ROW = _K + _BLK
_WPB = 4
_CPW = _M // _WPB


def _sc_body(pt_hbm, cen_hbm, out_hbm, px, py, pz, cx, cy, cz, row, res):
    wid = lax.axis_index("s") * 2 + lax.axis_index("c")
    b = wid // _WPB
    q = wid % _WPB

    pltpu.sync_copy(pt_hbm.at[pl.ds((b * 3 + 0) * _N, _N)], px)
    pltpu.sync_copy(pt_hbm.at[pl.ds((b * 3 + 1) * _N, _N)], py)
    pltpu.sync_copy(pt_hbm.at[pl.ds((b * 3 + 2) * _N, _N)], pz)
    pltpu.sync_copy(cen_hbm.at[pl.ds((b * 3 + 0) * _M + q * _CPW, _CPW)], cx)
    pltpu.sync_copy(cen_hbm.at[pl.ds((b * 3 + 1) * _M + q * _CPW, _CPW)], cy)
    pltpu.sync_copy(cen_hbm.at[pl.ds((b * 3 + 2) * _M + q * _CPW, _CPW)], cz)

    iota = lax.iota(jnp.int32, _L)
    sentinel = jnp.full((_L,), _N, jnp.int32)

    def _rne_bf16(v):
        u = plsc.bitcast(v, jnp.uint32)
        lsb = (u >> 16) & jnp.uint32(1)
        r = (u + jnp.uint32(0x7FFF) + lsb) & jnp.uint32(0xFFFF0000)
        return plsc.bitcast(r, jnp.float32)

    def precomp(j, carry):
        s = pl.ds(j * _L, _L)
        vx = px[s]
        vy = py[s]
        vz = pz[s]
        vsqp = (vx * vx + vy * vy) + vz * vz
        xb = plsc.bitcast(_rne_bf16(vx) * 2.0, jnp.uint32)
        yb = plsc.bitcast(_rne_bf16(vy) * 2.0, jnp.uint32)
        px[s] = plsc.bitcast(xb | (yb >> 16), jnp.float32)
        py[s] = _rne_bf16(vz) * 2.0
        pz[s] = vsqp
        return carry

    lax.fori_loop(0, _N // _L, precomp, 0)
    ones16 = jnp.ones((_L,), jnp.int32)

    def per_centroid(c, carry):
        cidx = jnp.full((_L,), c, jnp.int32)
        vcx = plsc.load_gather(cx, [cidx])
        vcy = plsc.load_gather(cy, [cidx])
        vcz = plsc.load_gather(cz, [cidx])
        sqc = (vcx * vcx + vcy * vcy) + vcz * vcz
        vcx = _rne_bf16(vcx)
        vcy = _rne_bf16(vcy)
        vcz = _rne_bf16(vcz)

        def cond(st):
            i, _, ps = st
            return (i < _NBLK) & (ps < _K - 1)

        def body(st):
            i, ptrv, _ = st
            base0 = i * _BLK
            masks = []
            prefixes = []
            counts = []
            for u in range(_U):
                s = pl.ds(base0 + u * _L, _L)
                w = plsc.bitcast(px[s], jnp.uint32)
                vx2 = plsc.bitcast(w & jnp.uint32(0xFFFF0000), jnp.float32)
                vy2 = plsc.bitcast(w << 16, jnp.float32)
                vz2 = py[s]
                vsqp = pz[s]
                cross2 = (vcx * vx2 + vcy * vy2) + vcz * vz2
                d2 = (sqc + vsqp) - cross2
                mask = d2 <= _RSQ
                masks.append(mask)
                prefixes.append(plsc.cumsum(ones16, mask=mask))
                counts.append(plsc.all_reduce_population_count(mask))
            for u in range(_U):
                pos = ptrv + prefixes[u]
                plsc.store_scatter(row, [pos], iota + (base0 + u * _L),
                                   mask=masks[u])
                ptrv = ptrv + counts[u]
            return i + 1, ptrv, jnp.max(ptrv)

        _, ptrv, _ = lax.while_loop(
            cond, body,
            (jnp.int32(0), jnp.full((_L,), -1, jnp.int32), jnp.int32(-1)),
        )

        row0 = row[pl.ds(0, _L)]
        first_s = jnp.min(jnp.where(iota == 0, row0, jnp.int32(2 ** 30)))
        padv = jnp.where(ptrv >= 0, jnp.full((_L,), first_s, jnp.int32), sentinel)
        for qq in range(_K // _L):
            lanes = iota + qq * _L
            vals = row[pl.ds(qq * _L, _L)]
            res[pl.ds(c * _K + qq * _L, _L)] = jnp.where(lanes <= ptrv, vals, padv)
        return carry

    lax.fori_loop(0, _CPW, per_centroid, 0)

    pltpu.sync_copy(res, out_hbm.at[pl.ds((b * _M + q * _CPW) * _K, _CPW * _K)])


@jax.jit
def kernel(pt_coordinates, centroids):
    f = pl.kernel(
        _sc_body,
        out_type=jax.ShapeDtypeStruct((_B * _M * _K,), jnp.int32),
        mesh=plsc.VectorSubcoreMesh(core_axis_name="c", subcore_axis_name="s"),
        compiler_params=pltpu.CompilerParams(needs_layout_passes=False),
        scratch_types=[
            pltpu.VMEM((_N,), jnp.float32),
            pltpu.VMEM((_N,), jnp.float32),
            pltpu.VMEM((_N,), jnp.float32),
            pltpu.VMEM((_CPW,), jnp.float32),
            pltpu.VMEM((_CPW,), jnp.float32),
            pltpu.VMEM((_CPW,), jnp.float32),
            pltpu.VMEM((_ROW,), jnp.int32),
            pltpu.VMEM((_CPW * _K,), jnp.int32),
        ],
    )
    out = f(pt_coordinates.reshape(-1), centroids.reshape(-1))
    return out.reshape(_B, _M, _K)
